# baseline (speedup 1.0000x reference)
import functools

import jax
import jax.numpy as jnp
from jax import lax
from jax.experimental import pallas as pl
from jax.experimental.pallas import tpu as pltpu
from jax.experimental.pallas import tpu_sc as plsc

N = 10000
E = 320000
DIN = 128
H = 256
HH = 128
NC = 2
NS = 16
LANES = 16
ZR = 624
ZTAIL = N - NS * ZR

BN_BLK = 1000
GN = N // BN_BLK
BE_BLK = 2000
GE = E // BE_BLK


def _sc_mesh():
  return plsc.VectorSubcoreMesh(
      core_axis_name="c", subcore_axis_name="s", num_cores=NC,
      num_subcores=NS)


def _make_edge_kernel(mode):
  BEDGE = 80
  per_tile = E // (NC * NS) if mode == 0 else E // NS
  nch = per_tile // BEDGE
  njj = nch // 2
  nz = ZR // BEDGE
  zrem = ZR % BEDGE

  @functools.partial(
      pl.kernel,
      out_type=jax.ShapeDtypeStruct((2 * N, HH), jnp.float32),
      mesh=_sc_mesh(),
      scratch_types=[
          pltpu.VMEM((BEDGE,), jnp.int32),
          pltpu.VMEM((BEDGE,), jnp.int32),
          pltpu.VMEM((BEDGE,), jnp.int32),
          pltpu.VMEM((BEDGE,), jnp.int32),
          pltpu.VMEM((BEDGE, HH), jnp.float32),
          pltpu.VMEM((BEDGE, HH), jnp.float32),
          pltpu.VMEM((BEDGE, HH), jnp.float32),
          pltpu.VMEM((BEDGE, HH), jnp.float32),
          pltpu.VMEM_SHARED((N, HH), jnp.float32),
          pltpu.SemaphoreType.DMA,
          pltpu.SemaphoreType.DMA,
          pltpu.SemaphoreType.DMA,
          pltpu.SemaphoreType.DMA,
          pltpu.SemaphoreType.DMA,
          pltpu.SemaphoreType.DMA,
          pltpu.SemaphoreType.DMA,
          pltpu.SemaphoreType.DMA,
      ],
  )
  def edge_kernel(src_hbm, dst_hbm, h_hbm, e_hbm, out_hbm,
                  sidx0, sidx1, didx0, didx1, evb0, evb1, rows0, rows1,
                  agg, ss0, ss1, sd0, sd1, se0, se1, sg0, sg1):
    c = lax.axis_index("c")
    s = lax.axis_index("s")
    sidx = (sidx0, sidx1)
    didx = (didx0, didx1)
    evb = (evb0, evb1)
    rows = (rows0, rows1)
    ssem = (ss0, ss1)
    dsem = (sd0, sd1)
    esem = (se0, se1)
    gsem = (sg0, sg1)

    zv = jnp.zeros((LANES,), jnp.float32)

    def zrow(j, carry):
      for k in range(HH // LANES):
        evb0[j, pl.ds(k * LANES, LANES)] = zv
      return carry

    lax.fori_loop(0, BEDGE, zrow, 0)
    row0 = s * ZR

    def zcopy(j, carry):
      pltpu.sync_copy(evb0, agg.at[pl.ds(row0 + j * BEDGE, BEDGE)])
      return carry

    lax.fori_loop(0, nz, zcopy, 0)
    if zrem:
      pltpu.sync_copy(evb0.at[pl.ds(0, zrem)],
                      agg.at[pl.ds(row0 + nz * BEDGE, zrem)])

    @pl.when(s == NS - 1)
    def _():
      pltpu.sync_copy(evb0.at[pl.ds(0, ZTAIL)],
                      agg.at[pl.ds(NS * ZR, ZTAIL)])

    plsc.subcore_barrier()

    if mode == 0:
      tile_base = c * (E // NC) + s * per_tile
      e_base = tile_base
      goff = None
    else:
      tile_base = s * per_tile
      e_base = c * E + tile_base
      goff = jnp.full((LANES,), c * N, jnp.int32)

    def lin_start(sl, chunk):
      base = tile_base + chunk * BEDGE
      pltpu.async_copy(src_hbm.at[pl.ds(base, BEDGE)], sidx[sl], ssem[sl])
      pltpu.async_copy(dst_hbm.at[pl.ds(base, BEDGE)], didx[sl], dsem[sl])
      pltpu.async_copy(e_hbm.at[pl.ds(e_base + chunk * BEDGE, BEDGE)],
                       evb[sl], esem[sl])

    def lin_wait_s(sl):
      pltpu.make_async_copy(src_hbm.at[pl.ds(0, BEDGE)], sidx[sl],
                            ssem[sl]).wait()

    def lin_wait_de(sl):
      pltpu.make_async_copy(dst_hbm.at[pl.ds(0, BEDGE)], didx[sl],
                            dsem[sl]).wait()
      pltpu.make_async_copy(e_hbm.at[pl.ds(0, BEDGE)], evb[sl],
                            esem[sl]).wait()

    def gath_start(sl):
      lin_wait_s(sl)
      if mode == 1:
        @plsc.parallel_loop(0, BEDGE // LANES, unroll=5)
        def _(j):
          slc = pl.ds(j * LANES, LANES)
          sidx[sl][slc] = sidx[sl][slc] + goff
      pltpu.async_copy(h_hbm.at[sidx[sl]], rows[sl], gsem[sl])

    def consume(sl):
      pltpu.make_async_copy(h_hbm.at[sidx[sl]], rows[sl], gsem[sl]).wait()
      lin_wait_de(sl)

      @plsc.parallel_loop(0, BEDGE, unroll=4)
      def _(j):
        for k in range(HH // LANES):
          slc = pl.ds(k * LANES, LANES)
          rows[sl][j, slc] = jnp.maximum(
              rows[sl][j, slc] + evb[sl][j, slc], 0.0)

      pltpu.sync_copy(rows[sl], agg.at[didx[sl]], add=True)

    lin_start(0, 0)
    gath_start(0)
    lin_start(1, 1)

    def step(j, carry):
      gath_start(1)
      consume(0)

      @pl.when(2 * j + 2 < nch)
      def _():
        lin_start(0, 2 * j + 2)

      consume(1)

      @pl.when(2 * j + 2 < nch)
      def _():
        gath_start(0)

      @pl.when(2 * j + 3 < nch)
      def _():
        lin_start(1, 2 * j + 3)

      return carry

    lax.fori_loop(0, njj, step, 0)
    if nch % 2:
      consume(0)
    plsc.subcore_barrier()

    def fcopy(j, carry):
      pltpu.sync_copy(agg.at[pl.ds(row0 + j * BEDGE, BEDGE)],
                      out_hbm.at[pl.ds(c * N + row0 + j * BEDGE, BEDGE)])
      return carry

    lax.fori_loop(0, nz, fcopy, 0)
    if zrem:
      pltpu.sync_copy(agg.at[pl.ds(row0 + nz * BEDGE, zrem)],
                      out_hbm.at[pl.ds(c * N + row0 + nz * BEDGE, zrem)])

    @pl.when(s == NS - 1)
    def _():
      pltpu.sync_copy(agg.at[pl.ds(NS * ZR, ZTAIL)],
                      out_hbm.at[pl.ds(c * N + NS * ZR, ZTAIL)])

  return edge_kernel


_edge_k0 = _make_edge_kernel(0)
_edge_k1 = _make_edge_kernel(1)


def _ke_body(ea_ref, w0, b0, w1, b1, w2, b2, e0_ref, e1_ref, e2_ref):
  ea = ea_ref[...]

  def mk(w, b):
    return jnp.dot(ea, w[...], preferred_element_type=jnp.float32) + b[...]

  e0_ref[...] = mk(w0, b0)
  r1 = mk(w1, b1)
  e1_ref[0] = r1[:, :HH]
  e1_ref[1] = r1[:, HH:]
  r2 = mk(w2, b2)
  e2_ref[0] = r2[:, :HH]
  e2_ref[1] = r2[:, HH:]


def _edge_transform(edge_attr, p0, p1, p2):
  full = lambda shape: pl.BlockSpec(shape, lambda i: tuple(0 for _ in shape))
  return pl.pallas_call(
      _ke_body,
      grid=(GE,),
      in_specs=[
          pl.BlockSpec((BE_BLK, 3), lambda i: (i, 0)),
          full((3, DIN)), full((1, DIN)),
          full((3, H)), full((1, H)),
          full((3, H)), full((1, H)),
      ],
      out_specs=[
          pl.BlockSpec((BE_BLK, DIN), lambda i: (i, 0)),
          pl.BlockSpec((2, BE_BLK, HH), lambda i: (0, i, 0)),
          pl.BlockSpec((2, BE_BLK, HH), lambda i: (0, i, 0)),
      ],
      out_shape=[
          jax.ShapeDtypeStruct((E, DIN), jnp.float32),
          jax.ShapeDtypeStruct((2, E, HH), jnp.float32),
          jax.ShapeDtypeStruct((2, E, HH), jnp.float32),
      ],
  )(edge_attr,
    p0['W_be'], p0['b_be'].reshape(1, DIN),
    p1['W_be'], p1['b_be'].reshape(1, H),
    p2['W_be'], p2['b_be'].reshape(1, H))


def _stats_update(i, u, sacc, qacc, st_ref):
  @pl.when(i == 0)
  def _():
    sacc[...] = jnp.zeros_like(sacc)
    qacc[...] = jnp.zeros_like(qacc)

  sacc[...] += jnp.sum(u, axis=0, keepdims=True)
  qacc[...] += jnp.sum(u * u, axis=0, keepdims=True)

  @pl.when(i == GN - 1)
  def _():
    st_ref[0:1, :] = sacc[...]
    st_ref[1:2, :] = qacc[...]


def _k1_body_l0(epsb, x_ref, agg_ref, w1, b1, u_ref, st_ref, sacc, qacc):
  i = pl.program_id(0)
  z = epsb[...] * x_ref[...] + agg_ref[0] + agg_ref[1]
  u = jnp.dot(z, w1[...], preferred_element_type=jnp.float32) + b1[...]
  u_ref[...] = u
  _stats_update(i, u, sacc, qacc, st_ref)


def _k1_body_lx(epsb, h_ref, agg_ref, w1, b1, u_ref, st_ref, sacc, qacc):
  i = pl.program_id(0)
  z0 = epsb[...] * h_ref[0] + agg_ref[0]
  z1 = epsb[...] * h_ref[1] + agg_ref[1]
  z = jnp.concatenate([z0, z1], axis=1)
  u = jnp.dot(z, w1[...], preferred_element_type=jnp.float32) + b1[...]
  u_ref[...] = u
  _stats_update(i, u, sacc, qacc, st_ref)


def _k1_call(layer0, epsb, hrep, agg, w1, b1):
  full = lambda shape: pl.BlockSpec(shape, lambda i: tuple(0 for _ in shape))
  din = DIN if layer0 else H
  hspec = (pl.BlockSpec((BN_BLK, DIN), lambda i: (i, 0)) if layer0
           else pl.BlockSpec((2, BN_BLK, HH), lambda i: (0, i, 0)))
  return pl.pallas_call(
      _k1_body_l0 if layer0 else _k1_body_lx,
      grid=(GN,),
      in_specs=[
          full((1, HH)),
          hspec,
          pl.BlockSpec((2, BN_BLK, HH), lambda i: (0, i, 0)),
          full((din, H)), full((1, H)),
      ],
      out_specs=[
          pl.BlockSpec((BN_BLK, H), lambda i: (i, 0)),
          pl.BlockSpec((2, H), lambda i: (0, 0)),
      ],
      out_shape=[
          jax.ShapeDtypeStruct((N, H), jnp.float32),
          jax.ShapeDtypeStruct((2, H), jnp.float32),
      ],
      scratch_shapes=[
          pltpu.VMEM((1, H), jnp.float32),
          pltpu.VMEM((1, H), jnp.float32),
      ],
  )(epsb, hrep, agg, w1, b1.reshape(1, H))


def _bn(u, st, g, be):
  mu = st[0:1, :] * (1.0 / N)
  var = st[1:2, :] * (1.0 / N) - mu * mu
  inv = lax.rsqrt(var + 1e-5)
  return (u - mu) * (inv * g[...]) + be[...]


def _k2_body(u1_ref, st_ref, g1, be1, w2, b2, u2_ref, st2_ref, sacc, qacc):
  i = pl.program_id(0)
  r = jnp.maximum(_bn(u1_ref[...], st_ref[...], g1, be1), 0.0)
  u2 = jnp.dot(r, w2[...], preferred_element_type=jnp.float32) + b2[...]
  u2_ref[...] = u2
  _stats_update(i, u2, sacc, qacc, st2_ref)


def _k2_call(u1, st, g1, be1, w2, b2):
  full = lambda shape: pl.BlockSpec(shape, lambda i: tuple(0 for _ in shape))
  return pl.pallas_call(
      _k2_body,
      grid=(GN,),
      in_specs=[
          pl.BlockSpec((BN_BLK, H), lambda i: (i, 0)),
          full((2, H)), full((1, H)), full((1, H)),
          full((H, H)), full((1, H)),
      ],
      out_specs=[
          pl.BlockSpec((BN_BLK, H), lambda i: (i, 0)),
          pl.BlockSpec((2, H), lambda i: (0, 0)),
      ],
      out_shape=[
          jax.ShapeDtypeStruct((N, H), jnp.float32),
          jax.ShapeDtypeStruct((2, H), jnp.float32),
      ],
      scratch_shapes=[
          pltpu.VMEM((1, H), jnp.float32),
          pltpu.VMEM((1, H), jnp.float32),
      ],
  )(u1, st, g1.reshape(1, H), be1.reshape(1, H), w2, b2.reshape(1, H))


def _k3_body_split(u2_ref, st_ref, g, be, out_ref):
  y = jnp.maximum(_bn(u2_ref[...], st_ref[...], g, be), 0.0)
  out_ref[0] = y[:, :HH]
  out_ref[1] = y[:, HH:]


def _k3_body_final(u2_ref, st_ref, g, be, out_ref):
  out_ref[...] = _bn(u2_ref[...], st_ref[...], g, be)


def _k3_call(final, u2, st, g, be):
  full = lambda shape: pl.BlockSpec(shape, lambda i: tuple(0 for _ in shape))
  out_specs = (pl.BlockSpec((BN_BLK, H), lambda i: (i, 0)) if final
               else pl.BlockSpec((2, BN_BLK, HH), lambda i: (0, i, 0)))
  out_shape = (jax.ShapeDtypeStruct((N, H), jnp.float32) if final
               else jax.ShapeDtypeStruct((2, N, HH), jnp.float32))
  return pl.pallas_call(
      _k3_body_final if final else _k3_body_split,
      grid=(GN,),
      in_specs=[
          pl.BlockSpec((BN_BLK, H), lambda i: (i, 0)),
          full((2, H)), full((1, H)), full((1, H)),
      ],
      out_specs=out_specs,
      out_shape=out_shape,
  )(u2, st, g.reshape(1, H), be.reshape(1, H))


def kernel(x, edge_index, edge_attr, params):
  src = edge_index[0]
  dst = edge_index[1]
  p0, p1, p2 = params['layer0'], params['layer1'], params['layer2']

  e0, e1, e2 = _edge_transform(edge_attr, p0, p1, p2)
  e1f = e1.reshape(2 * E, HH)
  e2f = e2.reshape(2 * E, HH)

  agg = _edge_k0(src, dst, x, e0).reshape(2, N, HH)
  epsb = jnp.broadcast_to(1.0 + p0['eps'], (1, HH))
  u1, st = _k1_call(True, epsb, x, agg, p0['W1'], p0['b1'])
  u2, st2 = _k2_call(u1, st, p0['g1'], p0['be1'], p0['W2'], p0['b2'])
  hcat = _k3_call(False, u2, st2, p0['g_out'], p0['b_out'])

  for p, ef, final in ((p1, e1f, False), (p2, e2f, True)):
    agg = _edge_k1(src, dst, hcat.reshape(2 * N, HH), ef).reshape(2, N, HH)
    epsb = jnp.broadcast_to(1.0 + p['eps'], (1, HH))
    u1, st = _k1_call(False, epsb, hcat, agg, p['W1'], p['b1'])
    u2, st2 = _k2_call(u1, st, p['g1'], p['be1'], p['W2'], p['b2'])
    out = _k3_call(final, u2, st2, p['g_out'], p['b_out'])
    if not final:
      hcat = out
  return out


# speedup vs baseline: 1.1630x; 1.1630x over previous
import functools

import jax
import jax.numpy as jnp
from jax import lax
from jax.experimental import pallas as pl
from jax.experimental.pallas import tpu as pltpu
from jax.experimental.pallas import tpu_sc as plsc

N = 10000
E = 320000
DIN = 128
H = 256
HH = 128
NC = 2
NS = 16
LANES = 16
ZR = 624
ZTAIL = N - NS * ZR

BN_BLK = 1000
GN = N // BN_BLK
BE_BLK = 2000
GE = E // BE_BLK


def _sc_mesh():
  return plsc.VectorSubcoreMesh(
      core_axis_name="c", subcore_axis_name="s", num_cores=NC,
      num_subcores=NS)


def _make_edge_kernel(mode):
  BEDGE = 80
  per_tile = E // (NC * NS) if mode == 0 else E // NS
  nch = per_tile // BEDGE
  njj = nch // 2
  nz = ZR // BEDGE
  zrem = ZR % BEDGE

  @functools.partial(
      pl.kernel,
      out_type=jax.ShapeDtypeStruct((2 * N, HH), jnp.float32),
      mesh=_sc_mesh(),
      scratch_types=[
          pltpu.VMEM((BEDGE,), jnp.int32),
          pltpu.VMEM((BEDGE,), jnp.int32),
          pltpu.VMEM((BEDGE,), jnp.int32),
          pltpu.VMEM((BEDGE,), jnp.int32),
          pltpu.VMEM((BEDGE, HH), jnp.float32),
          pltpu.VMEM((BEDGE, HH), jnp.float32),
          pltpu.VMEM((BEDGE, HH), jnp.float32),
          pltpu.VMEM((BEDGE, HH), jnp.float32),
          pltpu.VMEM_SHARED((N, HH), jnp.float32),
          pltpu.SemaphoreType.DMA,
          pltpu.SemaphoreType.DMA,
          pltpu.SemaphoreType.DMA,
          pltpu.SemaphoreType.DMA,
          pltpu.SemaphoreType.DMA,
          pltpu.SemaphoreType.DMA,
          pltpu.SemaphoreType.DMA,
          pltpu.SemaphoreType.DMA,
          pltpu.SemaphoreType.DMA,
          pltpu.SemaphoreType.DMA,
      ],
  )
  def edge_kernel(src_hbm, dst_hbm, h_hbm, e_hbm, out_hbm,
                  sidx0, sidx1, didx0, didx1, evb0, evb1, rows0, rows1,
                  agg, ss0, ss1, sd0, sd1, se0, se1, sg0, sg1, sx0, sx1):
    c = lax.axis_index("c")
    s = lax.axis_index("s")
    sidx = (sidx0, sidx1)
    didx = (didx0, didx1)
    evb = (evb0, evb1)
    rows = (rows0, rows1)
    ssem = (ss0, ss1)
    dsem = (sd0, sd1)
    esem = (se0, se1)
    gsem = (sg0, sg1)
    xsem = (sx0, sx1)

    zv = jnp.zeros((LANES,), jnp.float32)

    def zrow(j, carry):
      for k in range(HH // LANES):
        evb0[j, pl.ds(k * LANES, LANES)] = zv
      return carry

    lax.fori_loop(0, BEDGE, zrow, 0)
    row0 = s * ZR

    def zcopy(j, carry):
      pltpu.sync_copy(evb0, agg.at[pl.ds(row0 + j * BEDGE, BEDGE)])
      return carry

    lax.fori_loop(0, nz, zcopy, 0)
    if zrem:
      pltpu.sync_copy(evb0.at[pl.ds(0, zrem)],
                      agg.at[pl.ds(row0 + nz * BEDGE, zrem)])

    @pl.when(s == NS - 1)
    def _():
      pltpu.sync_copy(evb0.at[pl.ds(0, ZTAIL)],
                      agg.at[pl.ds(NS * ZR, ZTAIL)])

    plsc.subcore_barrier()

    if mode == 0:
      tile_base = c * (E // NC) + s * per_tile
      e_base = tile_base
      goff = None
    else:
      tile_base = s * per_tile
      e_base = c * E + tile_base
      goff = jnp.full((LANES,), c * N, jnp.int32)

    def lin_start(sl, chunk):
      base = tile_base + chunk * BEDGE
      pltpu.async_copy(src_hbm.at[pl.ds(base, BEDGE)], sidx[sl], ssem[sl])
      pltpu.async_copy(e_hbm.at[pl.ds(e_base + chunk * BEDGE, BEDGE)],
                       evb[sl], esem[sl])

    def lin_wait_s(sl):
      pltpu.make_async_copy(src_hbm.at[pl.ds(0, BEDGE)], sidx[sl],
                            ssem[sl]).wait()

    def lin_wait_de(sl):
      pltpu.make_async_copy(dst_hbm.at[pl.ds(0, BEDGE)], didx[sl],
                            dsem[sl]).wait()
      pltpu.make_async_copy(e_hbm.at[pl.ds(0, BEDGE)], evb[sl],
                            esem[sl]).wait()

    def scat_wait(sl):
      pltpu.make_async_copy(rows[sl], agg.at[didx[sl]], xsem[sl]).wait()

    def gath_start(sl, chunk, first):
      if not first:
        scat_wait(sl)
      base = tile_base + chunk * BEDGE
      pltpu.async_copy(dst_hbm.at[pl.ds(base, BEDGE)], didx[sl], dsem[sl])
      lin_wait_s(sl)
      if mode == 1:
        @plsc.parallel_loop(0, BEDGE // LANES, unroll=5)
        def _(j):
          slc = pl.ds(j * LANES, LANES)
          sidx[sl][slc] = sidx[sl][slc] + goff
      pltpu.async_copy(h_hbm.at[sidx[sl]], rows[sl], gsem[sl])

    def consume(sl):
      pltpu.make_async_copy(h_hbm.at[sidx[sl]], rows[sl], gsem[sl]).wait()
      lin_wait_de(sl)

      @plsc.parallel_loop(0, BEDGE, unroll=4)
      def _(j):
        for k in range(HH // LANES):
          slc = pl.ds(k * LANES, LANES)
          rows[sl][j, slc] = jnp.maximum(
              rows[sl][j, slc] + evb[sl][j, slc], 0.0)

      pltpu.async_copy(rows[sl], agg.at[didx[sl]], xsem[sl], add=True)

    def step(j, first):
      gath_start(1, 2 * j + 1, first)
      consume(0)

      @pl.when(2 * j + 2 < nch)
      def _():
        lin_start(0, 2 * j + 2)

      consume(1)

      @pl.when(2 * j + 2 < nch)
      def _():
        gath_start(0, 2 * j + 2, False)

      @pl.when(2 * j + 3 < nch)
      def _():
        lin_start(1, 2 * j + 3)

    lin_start(0, 0)
    gath_start(0, 0, True)
    lin_start(1, 1)
    step(0, True)

    def stepc(j, carry):
      step(j, False)
      return carry

    lax.fori_loop(1, njj, stepc, 0)
    if nch % 2:
      consume(0)
    scat_wait(0)
    scat_wait(1)
    plsc.subcore_barrier()

    def fcopy(j, carry):
      pltpu.sync_copy(agg.at[pl.ds(row0 + j * BEDGE, BEDGE)],
                      out_hbm.at[pl.ds(c * N + row0 + j * BEDGE, BEDGE)])
      return carry

    lax.fori_loop(0, nz, fcopy, 0)
    if zrem:
      pltpu.sync_copy(agg.at[pl.ds(row0 + nz * BEDGE, zrem)],
                      out_hbm.at[pl.ds(c * N + row0 + nz * BEDGE, zrem)])

    @pl.when(s == NS - 1)
    def _():
      pltpu.sync_copy(agg.at[pl.ds(NS * ZR, ZTAIL)],
                      out_hbm.at[pl.ds(c * N + NS * ZR, ZTAIL)])

  return edge_kernel


_edge_k0 = _make_edge_kernel(0)
_edge_k1 = _make_edge_kernel(1)


def _ke_body(ea_ref, w0, b0, w1, b1, w2, b2, e0_ref, e1_ref, e2_ref):
  ea = ea_ref[...]

  def mk(w, b):
    return jnp.dot(ea, w[...], preferred_element_type=jnp.float32) + b[...]

  e0_ref[...] = mk(w0, b0)
  r1 = mk(w1, b1)
  e1_ref[0] = r1[:, :HH]
  e1_ref[1] = r1[:, HH:]
  r2 = mk(w2, b2)
  e2_ref[0] = r2[:, :HH]
  e2_ref[1] = r2[:, HH:]


def _edge_transform(edge_attr, p0, p1, p2):
  full = lambda shape: pl.BlockSpec(shape, lambda i: tuple(0 for _ in shape))
  return pl.pallas_call(
      _ke_body,
      grid=(GE,),
      in_specs=[
          pl.BlockSpec((BE_BLK, 3), lambda i: (i, 0)),
          full((3, DIN)), full((1, DIN)),
          full((3, H)), full((1, H)),
          full((3, H)), full((1, H)),
      ],
      out_specs=[
          pl.BlockSpec((BE_BLK, DIN), lambda i: (i, 0)),
          pl.BlockSpec((2, BE_BLK, HH), lambda i: (0, i, 0)),
          pl.BlockSpec((2, BE_BLK, HH), lambda i: (0, i, 0)),
      ],
      out_shape=[
          jax.ShapeDtypeStruct((E, DIN), jnp.float32),
          jax.ShapeDtypeStruct((2, E, HH), jnp.float32),
          jax.ShapeDtypeStruct((2, E, HH), jnp.float32),
      ],
  )(edge_attr,
    p0['W_be'], p0['b_be'].reshape(1, DIN),
    p1['W_be'], p1['b_be'].reshape(1, H),
    p2['W_be'], p2['b_be'].reshape(1, H))


def _stats_update(i, u, sacc, qacc, st_ref):
  @pl.when(i == 0)
  def _():
    sacc[...] = jnp.zeros_like(sacc)
    qacc[...] = jnp.zeros_like(qacc)

  sacc[...] += jnp.sum(u, axis=0, keepdims=True)
  qacc[...] += jnp.sum(u * u, axis=0, keepdims=True)

  @pl.when(i == GN - 1)
  def _():
    st_ref[0:1, :] = sacc[...]
    st_ref[1:2, :] = qacc[...]


def _k1_body_l0(epsb, x_ref, agg_ref, w1, b1, u_ref, st_ref, sacc, qacc):
  i = pl.program_id(0)
  z = epsb[...] * x_ref[...] + agg_ref[0] + agg_ref[1]
  u = jnp.dot(z, w1[...], preferred_element_type=jnp.float32) + b1[...]
  u_ref[...] = u
  _stats_update(i, u, sacc, qacc, st_ref)


def _k1_body_lx(epsb, h_ref, agg_ref, w1, b1, u_ref, st_ref, sacc, qacc):
  i = pl.program_id(0)
  z0 = epsb[...] * h_ref[0] + agg_ref[0]
  z1 = epsb[...] * h_ref[1] + agg_ref[1]
  z = jnp.concatenate([z0, z1], axis=1)
  u = jnp.dot(z, w1[...], preferred_element_type=jnp.float32) + b1[...]
  u_ref[...] = u
  _stats_update(i, u, sacc, qacc, st_ref)


def _k1_call(layer0, epsb, hrep, agg, w1, b1):
  full = lambda shape: pl.BlockSpec(shape, lambda i: tuple(0 for _ in shape))
  din = DIN if layer0 else H
  hspec = (pl.BlockSpec((BN_BLK, DIN), lambda i: (i, 0)) if layer0
           else pl.BlockSpec((2, BN_BLK, HH), lambda i: (0, i, 0)))
  return pl.pallas_call(
      _k1_body_l0 if layer0 else _k1_body_lx,
      grid=(GN,),
      in_specs=[
          full((1, HH)),
          hspec,
          pl.BlockSpec((2, BN_BLK, HH), lambda i: (0, i, 0)),
          full((din, H)), full((1, H)),
      ],
      out_specs=[
          pl.BlockSpec((BN_BLK, H), lambda i: (i, 0)),
          pl.BlockSpec((2, H), lambda i: (0, 0)),
      ],
      out_shape=[
          jax.ShapeDtypeStruct((N, H), jnp.float32),
          jax.ShapeDtypeStruct((2, H), jnp.float32),
      ],
      scratch_shapes=[
          pltpu.VMEM((1, H), jnp.float32),
          pltpu.VMEM((1, H), jnp.float32),
      ],
  )(epsb, hrep, agg, w1, b1.reshape(1, H))


def _bn(u, st, g, be):
  mu = st[0:1, :] * (1.0 / N)
  var = st[1:2, :] * (1.0 / N) - mu * mu
  inv = lax.rsqrt(var + 1e-5)
  return (u - mu) * (inv * g[...]) + be[...]


def _k2_body(u1_ref, st_ref, g1, be1, w2, b2, u2_ref, st2_ref, sacc, qacc):
  i = pl.program_id(0)
  r = jnp.maximum(_bn(u1_ref[...], st_ref[...], g1, be1), 0.0)
  u2 = jnp.dot(r, w2[...], preferred_element_type=jnp.float32) + b2[...]
  u2_ref[...] = u2
  _stats_update(i, u2, sacc, qacc, st2_ref)


def _k2_call(u1, st, g1, be1, w2, b2):
  full = lambda shape: pl.BlockSpec(shape, lambda i: tuple(0 for _ in shape))
  return pl.pallas_call(
      _k2_body,
      grid=(GN,),
      in_specs=[
          pl.BlockSpec((BN_BLK, H), lambda i: (i, 0)),
          full((2, H)), full((1, H)), full((1, H)),
          full((H, H)), full((1, H)),
      ],
      out_specs=[
          pl.BlockSpec((BN_BLK, H), lambda i: (i, 0)),
          pl.BlockSpec((2, H), lambda i: (0, 0)),
      ],
      out_shape=[
          jax.ShapeDtypeStruct((N, H), jnp.float32),
          jax.ShapeDtypeStruct((2, H), jnp.float32),
      ],
      scratch_shapes=[
          pltpu.VMEM((1, H), jnp.float32),
          pltpu.VMEM((1, H), jnp.float32),
      ],
  )(u1, st, g1.reshape(1, H), be1.reshape(1, H), w2, b2.reshape(1, H))


def _k3_body_split(u2_ref, st_ref, g, be, out_ref):
  y = jnp.maximum(_bn(u2_ref[...], st_ref[...], g, be), 0.0)
  out_ref[0] = y[:, :HH]
  out_ref[1] = y[:, HH:]


def _k3_body_final(u2_ref, st_ref, g, be, out_ref):
  out_ref[...] = _bn(u2_ref[...], st_ref[...], g, be)


def _k3_call(final, u2, st, g, be):
  full = lambda shape: pl.BlockSpec(shape, lambda i: tuple(0 for _ in shape))
  out_specs = (pl.BlockSpec((BN_BLK, H), lambda i: (i, 0)) if final
               else pl.BlockSpec((2, BN_BLK, HH), lambda i: (0, i, 0)))
  out_shape = (jax.ShapeDtypeStruct((N, H), jnp.float32) if final
               else jax.ShapeDtypeStruct((2, N, HH), jnp.float32))
  return pl.pallas_call(
      _k3_body_final if final else _k3_body_split,
      grid=(GN,),
      in_specs=[
          pl.BlockSpec((BN_BLK, H), lambda i: (i, 0)),
          full((2, H)), full((1, H)), full((1, H)),
      ],
      out_specs=out_specs,
      out_shape=out_shape,
  )(u2, st, g.reshape(1, H), be.reshape(1, H))


def kernel(x, edge_index, edge_attr, params):
  src = edge_index[0]
  dst = edge_index[1]
  p0, p1, p2 = params['layer0'], params['layer1'], params['layer2']

  e0, e1, e2 = _edge_transform(edge_attr, p0, p1, p2)
  e1f = e1.reshape(2 * E, HH)
  e2f = e2.reshape(2 * E, HH)

  agg = _edge_k0(src, dst, x, e0).reshape(2, N, HH)
  epsb = jnp.broadcast_to(1.0 + p0['eps'], (1, HH))
  u1, st = _k1_call(True, epsb, x, agg, p0['W1'], p0['b1'])
  u2, st2 = _k2_call(u1, st, p0['g1'], p0['be1'], p0['W2'], p0['b2'])
  hcat = _k3_call(False, u2, st2, p0['g_out'], p0['b_out'])

  for p, ef, final in ((p1, e1f, False), (p2, e2f, True)):
    agg = _edge_k1(src, dst, hcat.reshape(2 * N, HH), ef).reshape(2, N, HH)
    epsb = jnp.broadcast_to(1.0 + p['eps'], (1, HH))
    u1, st = _k1_call(False, epsb, hcat, agg, p['W1'], p['b1'])
    u2, st2 = _k2_call(u1, st, p['g1'], p['be1'], p['W2'], p['b2'])
    out = _k3_call(final, u2, st2, p['g_out'], p['b_out'])
    if not final:
      hcat = out
  return out


# speedup vs baseline: 1.1941x; 1.0267x over previous
import functools

import jax
import jax.numpy as jnp
from jax import lax
from jax.experimental import pallas as pl
from jax.experimental.pallas import tpu as pltpu
from jax.experimental.pallas import tpu_sc as plsc

N = 10000
E = 320000
DIN = 128
H = 256
HH = 128
NC = 2
NS = 16
LANES = 16
ZR = 624
ZTAIL = N - NS * ZR

BN_BLK = 2000
GN = N // BN_BLK
BE_BLK = 4000
GE = E // BE_BLK


def _sc_mesh():
  return plsc.VectorSubcoreMesh(
      core_axis_name="c", subcore_axis_name="s", num_cores=NC,
      num_subcores=NS)


def _make_edge_kernel(mode):
  BEDGE = 80
  per_tile = E // (NC * NS) if mode == 0 else E // NS
  nch = per_tile // BEDGE
  njj = nch // 2
  nz = ZR // BEDGE
  zrem = ZR % BEDGE

  @functools.partial(
      pl.kernel,
      out_type=jax.ShapeDtypeStruct((2 * N, HH), jnp.float32),
      mesh=_sc_mesh(),
      scratch_types=[
          pltpu.VMEM((BEDGE,), jnp.int32),
          pltpu.VMEM((BEDGE,), jnp.int32),
          pltpu.VMEM((BEDGE,), jnp.int32),
          pltpu.VMEM((BEDGE,), jnp.int32),
          pltpu.VMEM((BEDGE, HH), jnp.float32),
          pltpu.VMEM((BEDGE, HH), jnp.float32),
          pltpu.VMEM((BEDGE, HH), jnp.float32),
          pltpu.VMEM((BEDGE, HH), jnp.float32),
          pltpu.VMEM_SHARED((N, HH), jnp.float32),
          pltpu.SemaphoreType.DMA,
          pltpu.SemaphoreType.DMA,
          pltpu.SemaphoreType.DMA,
          pltpu.SemaphoreType.DMA,
          pltpu.SemaphoreType.DMA,
          pltpu.SemaphoreType.DMA,
          pltpu.SemaphoreType.DMA,
          pltpu.SemaphoreType.DMA,
          pltpu.SemaphoreType.DMA,
          pltpu.SemaphoreType.DMA,
      ],
  )
  def edge_kernel(src_hbm, dst_hbm, h_hbm, e_hbm, out_hbm,
                  sidx0, sidx1, didx0, didx1, evb0, evb1, rows0, rows1,
                  agg, ss0, ss1, sd0, sd1, se0, se1, sg0, sg1, sx0, sx1):
    c = lax.axis_index("c")
    s = lax.axis_index("s")
    sidx = (sidx0, sidx1)
    didx = (didx0, didx1)
    evb = (evb0, evb1)
    rows = (rows0, rows1)
    ssem = (ss0, ss1)
    dsem = (sd0, sd1)
    esem = (se0, se1)
    gsem = (sg0, sg1)
    xsem = (sx0, sx1)

    zv = jnp.zeros((LANES,), jnp.float32)

    def zrow(j, carry):
      for k in range(HH // LANES):
        evb0[j, pl.ds(k * LANES, LANES)] = zv
      return carry

    lax.fori_loop(0, BEDGE, zrow, 0)
    row0 = s * ZR

    def zcopy(j, carry):
      pltpu.sync_copy(evb0, agg.at[pl.ds(row0 + j * BEDGE, BEDGE)])
      return carry

    lax.fori_loop(0, nz, zcopy, 0)
    if zrem:
      pltpu.sync_copy(evb0.at[pl.ds(0, zrem)],
                      agg.at[pl.ds(row0 + nz * BEDGE, zrem)])

    @pl.when(s == NS - 1)
    def _():
      pltpu.sync_copy(evb0.at[pl.ds(0, ZTAIL)],
                      agg.at[pl.ds(NS * ZR, ZTAIL)])

    plsc.subcore_barrier()

    if mode == 0:
      tile_base = c * (E // NC) + s * per_tile
      e_base = tile_base
      goff = None
    else:
      tile_base = s * per_tile
      e_base = c * E + tile_base
      goff = jnp.full((LANES,), c * N, jnp.int32)

    def lin_start(sl, chunk):
      base = tile_base + chunk * BEDGE
      pltpu.async_copy(src_hbm.at[pl.ds(base, BEDGE)], sidx[sl], ssem[sl])
      pltpu.async_copy(e_hbm.at[pl.ds(e_base + chunk * BEDGE, BEDGE)],
                       evb[sl], esem[sl])

    def lin_wait_s(sl):
      pltpu.make_async_copy(src_hbm.at[pl.ds(0, BEDGE)], sidx[sl],
                            ssem[sl]).wait()

    def lin_wait_de(sl):
      pltpu.make_async_copy(dst_hbm.at[pl.ds(0, BEDGE)], didx[sl],
                            dsem[sl]).wait()
      pltpu.make_async_copy(e_hbm.at[pl.ds(0, BEDGE)], evb[sl],
                            esem[sl]).wait()

    def scat_wait(sl):
      pltpu.make_async_copy(rows[sl], agg.at[didx[sl]], xsem[sl]).wait()

    def gath_start(sl, chunk, first):
      if not first:
        scat_wait(sl)
      base = tile_base + chunk * BEDGE
      pltpu.async_copy(dst_hbm.at[pl.ds(base, BEDGE)], didx[sl], dsem[sl])
      lin_wait_s(sl)
      if mode == 1:
        @plsc.parallel_loop(0, BEDGE // LANES, unroll=5)
        def _(j):
          slc = pl.ds(j * LANES, LANES)
          sidx[sl][slc] = sidx[sl][slc] + goff
      pltpu.async_copy(h_hbm.at[sidx[sl]], rows[sl], gsem[sl])

    def consume(sl):
      pltpu.make_async_copy(h_hbm.at[sidx[sl]], rows[sl], gsem[sl]).wait()
      lin_wait_de(sl)

      @plsc.parallel_loop(0, BEDGE, unroll=4)
      def _(j):
        for k in range(HH // LANES):
          slc = pl.ds(k * LANES, LANES)
          rows[sl][j, slc] = jnp.maximum(
              rows[sl][j, slc] + evb[sl][j, slc], 0.0)

      pltpu.async_copy(rows[sl], agg.at[didx[sl]], xsem[sl], add=True)

    def step(j, first):
      gath_start(1, 2 * j + 1, first)
      consume(0)

      @pl.when(2 * j + 2 < nch)
      def _():
        lin_start(0, 2 * j + 2)

      consume(1)

      @pl.when(2 * j + 2 < nch)
      def _():
        gath_start(0, 2 * j + 2, False)

      @pl.when(2 * j + 3 < nch)
      def _():
        lin_start(1, 2 * j + 3)

    lin_start(0, 0)
    gath_start(0, 0, True)
    lin_start(1, 1)
    step(0, True)

    def stepc(j, carry):
      step(j, False)
      return carry

    lax.fori_loop(1, njj, stepc, 0)
    if nch % 2:
      consume(0)
    scat_wait(0)
    scat_wait(1)
    plsc.subcore_barrier()

    def fcopy(j, carry):
      pltpu.sync_copy(agg.at[pl.ds(row0 + j * BEDGE, BEDGE)],
                      out_hbm.at[pl.ds(c * N + row0 + j * BEDGE, BEDGE)])
      return carry

    lax.fori_loop(0, nz, fcopy, 0)
    if zrem:
      pltpu.sync_copy(agg.at[pl.ds(row0 + nz * BEDGE, zrem)],
                      out_hbm.at[pl.ds(c * N + row0 + nz * BEDGE, zrem)])

    @pl.when(s == NS - 1)
    def _():
      pltpu.sync_copy(agg.at[pl.ds(NS * ZR, ZTAIL)],
                      out_hbm.at[pl.ds(c * N + NS * ZR, ZTAIL)])

  return edge_kernel


_edge_k0 = _make_edge_kernel(0)
_edge_k1 = _make_edge_kernel(1)


def _ke_body(ea_ref, w0, b0, w1, b1, w2, b2, e0_ref, e1_ref, e2_ref):
  ea = ea_ref[...]

  def mk(w, b):
    return jnp.dot(ea, w[...], preferred_element_type=jnp.float32) + b[...]

  e0_ref[...] = mk(w0, b0)
  r1 = mk(w1, b1)
  e1_ref[0] = r1[:, :HH]
  e1_ref[1] = r1[:, HH:]
  r2 = mk(w2, b2)
  e2_ref[0] = r2[:, :HH]
  e2_ref[1] = r2[:, HH:]


def _edge_transform(edge_attr, p0, p1, p2):
  full = lambda shape: pl.BlockSpec(shape, lambda i: tuple(0 for _ in shape))
  return pl.pallas_call(
      _ke_body,
      grid=(GE,),
      in_specs=[
          pl.BlockSpec((BE_BLK, 3), lambda i: (i, 0)),
          full((3, DIN)), full((1, DIN)),
          full((3, H)), full((1, H)),
          full((3, H)), full((1, H)),
      ],
      out_specs=[
          pl.BlockSpec((BE_BLK, DIN), lambda i: (i, 0)),
          pl.BlockSpec((2, BE_BLK, HH), lambda i: (0, i, 0)),
          pl.BlockSpec((2, BE_BLK, HH), lambda i: (0, i, 0)),
      ],
      out_shape=[
          jax.ShapeDtypeStruct((E, DIN), jnp.float32),
          jax.ShapeDtypeStruct((2, E, HH), jnp.float32),
          jax.ShapeDtypeStruct((2, E, HH), jnp.float32),
      ],
  )(edge_attr,
    p0['W_be'], p0['b_be'].reshape(1, DIN),
    p1['W_be'], p1['b_be'].reshape(1, H),
    p2['W_be'], p2['b_be'].reshape(1, H))


def _stats_update(i, u, sacc, qacc, st_ref):
  @pl.when(i == 0)
  def _():
    sacc[...] = jnp.zeros_like(sacc)
    qacc[...] = jnp.zeros_like(qacc)

  sacc[...] += jnp.sum(u, axis=0, keepdims=True)
  qacc[...] += jnp.sum(u * u, axis=0, keepdims=True)

  @pl.when(i == GN - 1)
  def _():
    st_ref[0:1, :] = sacc[...]
    st_ref[1:2, :] = qacc[...]


def _k1_body_l0(epsb, x_ref, agg_ref, w1, b1, u_ref, st_ref, sacc, qacc):
  i = pl.program_id(0)
  z = epsb[...] * x_ref[...] + agg_ref[0] + agg_ref[1]
  u = jnp.dot(z, w1[...], preferred_element_type=jnp.float32) + b1[...]
  u_ref[...] = u
  _stats_update(i, u, sacc, qacc, st_ref)


def _k1_body_lx(epsb, h_ref, agg_ref, w1, b1, u_ref, st_ref, sacc, qacc):
  i = pl.program_id(0)
  z0 = epsb[...] * h_ref[0] + agg_ref[0]
  z1 = epsb[...] * h_ref[1] + agg_ref[1]
  z = jnp.concatenate([z0, z1], axis=1)
  u = jnp.dot(z, w1[...], preferred_element_type=jnp.float32) + b1[...]
  u_ref[...] = u
  _stats_update(i, u, sacc, qacc, st_ref)


def _k1_call(layer0, epsb, hrep, agg, w1, b1):
  full = lambda shape: pl.BlockSpec(shape, lambda i: tuple(0 for _ in shape))
  din = DIN if layer0 else H
  hspec = (pl.BlockSpec((BN_BLK, DIN), lambda i: (i, 0)) if layer0
           else pl.BlockSpec((2, BN_BLK, HH), lambda i: (0, i, 0)))
  return pl.pallas_call(
      _k1_body_l0 if layer0 else _k1_body_lx,
      grid=(GN,),
      in_specs=[
          full((1, HH)),
          hspec,
          pl.BlockSpec((2, BN_BLK, HH), lambda i: (0, i, 0)),
          full((din, H)), full((1, H)),
      ],
      out_specs=[
          pl.BlockSpec((BN_BLK, H), lambda i: (i, 0)),
          pl.BlockSpec((2, H), lambda i: (0, 0)),
      ],
      out_shape=[
          jax.ShapeDtypeStruct((N, H), jnp.float32),
          jax.ShapeDtypeStruct((2, H), jnp.float32),
      ],
      scratch_shapes=[
          pltpu.VMEM((1, H), jnp.float32),
          pltpu.VMEM((1, H), jnp.float32),
      ],
  )(epsb, hrep, agg, w1, b1.reshape(1, H))


def _bn(u, st, g, be):
  mu = st[0:1, :] * (1.0 / N)
  var = st[1:2, :] * (1.0 / N) - mu * mu
  inv = lax.rsqrt(var + 1e-5)
  return (u - mu) * (inv * g[...]) + be[...]


def _k2_body(u1_ref, st_ref, g1, be1, w2, b2, u2_ref, st2_ref, sacc, qacc):
  i = pl.program_id(0)
  r = jnp.maximum(_bn(u1_ref[...], st_ref[...], g1, be1), 0.0)
  u2 = jnp.dot(r, w2[...], preferred_element_type=jnp.float32) + b2[...]
  u2_ref[...] = u2
  _stats_update(i, u2, sacc, qacc, st2_ref)


def _k2_call(u1, st, g1, be1, w2, b2):
  full = lambda shape: pl.BlockSpec(shape, lambda i: tuple(0 for _ in shape))
  return pl.pallas_call(
      _k2_body,
      grid=(GN,),
      in_specs=[
          pl.BlockSpec((BN_BLK, H), lambda i: (i, 0)),
          full((2, H)), full((1, H)), full((1, H)),
          full((H, H)), full((1, H)),
      ],
      out_specs=[
          pl.BlockSpec((BN_BLK, H), lambda i: (i, 0)),
          pl.BlockSpec((2, H), lambda i: (0, 0)),
      ],
      out_shape=[
          jax.ShapeDtypeStruct((N, H), jnp.float32),
          jax.ShapeDtypeStruct((2, H), jnp.float32),
      ],
      scratch_shapes=[
          pltpu.VMEM((1, H), jnp.float32),
          pltpu.VMEM((1, H), jnp.float32),
      ],
  )(u1, st, g1.reshape(1, H), be1.reshape(1, H), w2, b2.reshape(1, H))


def _k3_body_split(u2_ref, st_ref, g, be, out_ref):
  y = jnp.maximum(_bn(u2_ref[...], st_ref[...], g, be), 0.0)
  out_ref[0] = y[:, :HH]
  out_ref[1] = y[:, HH:]


def _k3_body_final(u2_ref, st_ref, g, be, out_ref):
  out_ref[...] = _bn(u2_ref[...], st_ref[...], g, be)


def _k3_call(final, u2, st, g, be):
  full = lambda shape: pl.BlockSpec(shape, lambda i: tuple(0 for _ in shape))
  out_specs = (pl.BlockSpec((BN_BLK, H), lambda i: (i, 0)) if final
               else pl.BlockSpec((2, BN_BLK, HH), lambda i: (0, i, 0)))
  out_shape = (jax.ShapeDtypeStruct((N, H), jnp.float32) if final
               else jax.ShapeDtypeStruct((2, N, HH), jnp.float32))
  return pl.pallas_call(
      _k3_body_final if final else _k3_body_split,
      grid=(GN,),
      in_specs=[
          pl.BlockSpec((BN_BLK, H), lambda i: (i, 0)),
          full((2, H)), full((1, H)), full((1, H)),
      ],
      out_specs=out_specs,
      out_shape=out_shape,
  )(u2, st, g.reshape(1, H), be.reshape(1, H))


def kernel(x, edge_index, edge_attr, params):
  src = edge_index[0]
  dst = edge_index[1]
  p0, p1, p2 = params['layer0'], params['layer1'], params['layer2']

  e0, e1, e2 = _edge_transform(edge_attr, p0, p1, p2)
  e1f = e1.reshape(2 * E, HH)
  e2f = e2.reshape(2 * E, HH)

  agg = _edge_k0(src, dst, x, e0).reshape(2, N, HH)
  epsb = jnp.broadcast_to(1.0 + p0['eps'], (1, HH))
  u1, st = _k1_call(True, epsb, x, agg, p0['W1'], p0['b1'])
  u2, st2 = _k2_call(u1, st, p0['g1'], p0['be1'], p0['W2'], p0['b2'])
  hcat = _k3_call(False, u2, st2, p0['g_out'], p0['b_out'])

  for p, ef, final in ((p1, e1f, False), (p2, e2f, True)):
    agg = _edge_k1(src, dst, hcat.reshape(2 * N, HH), ef).reshape(2, N, HH)
    epsb = jnp.broadcast_to(1.0 + p['eps'], (1, HH))
    u1, st = _k1_call(False, epsb, hcat, agg, p['W1'], p['b1'])
    u2, st2 = _k2_call(u1, st, p['g1'], p['be1'], p['W2'], p['b2'])
    out = _k3_call(final, u2, st2, p['g_out'], p['b_out'])
    if not final:
      hcat = out
  return out
